# Initial kernel scaffold; baseline (speedup 1.0000x reference)
import jax
import jax.numpy as jnp
from jax.experimental import pallas as pl


def kernel(x, edge_index, W1, b1, W2, b2):
    raise NotImplementedError("write your pallas kernel here")


# speedup vs baseline: 25.0922x; 25.0922x over previous
import functools

import jax
import jax.numpy as jnp
from jax import lax
from jax.experimental import pallas as pl
from jax.experimental.pallas import tpu as pltpu
from jax.experimental.pallas import tpu_sc as plsc

N = 50000
E = 1600000
D_IN = 1433
D_HID = 16
D_OUT = 7
D_OUT_PAD = 8

NC = 2
NS = 16
NT = NC * NS

CHUNK = 128
CPT = (E + NT * CHUNK - 1) // (NT * CHUNK)
E_PAD = NT * CPT * CHUNK

N_PAD = 50048
ROWS_A = N_PAD // NS
ROWS_LAST = N - (NS - 1) * ROWS_A

ROW_BLK = 400
GRID = N // ROW_BLK

_MESH = plsc.VectorSubcoreMesh(
    core_axis_name="c", subcore_axis_name="s", num_cores=NC, num_subcores=NS
)


ZBUF = ROWS_A + 8


def _hist_body(dsts_hbm, out_hbm, dst_idx, ones, zbuf, deg_acc, sem):
    c = lax.axis_index("c")
    s = lax.axis_index("s")
    for i in range(CHUNK // 16):
        ones[pl.ds(i * 16, 16)] = jnp.ones((16,), jnp.float32)
    for i in range(ZBUF // 16):
        zbuf[pl.ds(i * 16, 16)] = jnp.zeros((16,), jnp.float32)
    pltpu.sync_copy(zbuf.at[pl.ds(0, ROWS_A)],
                    deg_acc.at[pl.ds(s * ROWS_A, ROWS_A)])
    pltpu.sync_copy(dsts_hbm.at[c, s], dst_idx)
    plsc.subcore_barrier()

    def body(j, carry):
        pltpu.sync_copy(ones, deg_acc.at[dst_idx.at[j]], add=True)
        return carry

    lax.fori_loop(0, CPT, body, 0)
    plsc.subcore_barrier()
    pltpu.sync_copy(deg_acc.at[pl.ds(s * ROWS_A, ROWS_A)],
                    zbuf.at[pl.ds(0, ROWS_A)])
    pltpu.sync_copy(zbuf.at[pl.ds(0, ROWS_A)],
                    out_hbm.at[pl.ds(c * N_PAD + s * ROWS_A, ROWS_A)])


_hist = functools.partial(
    pl.kernel,
    out_type=pltpu.HBM((NC * N_PAD,), jnp.float32),
    mesh=_MESH,
    compiler_params=pltpu.CompilerParams(use_tc_tiling_on_sc=False),
    scratch_types=[
        pltpu.VMEM((CPT, CHUNK), jnp.int32),
        pltpu.VMEM((CHUNK,), jnp.float32),
        pltpu.VMEM((ZBUF,), jnp.float32),
        pltpu.VMEM_SHARED((N_PAD,), jnp.float32),
        pltpu.SemaphoreType.DMA,
    ],
)(_hist_body)


def _make_agg(d):
    def body(t_hbm, srcs_hbm, dsts_hbm, zeros_hbm, out_hbm,
             src_idx, dst_idx, rows, acc_spm, sem):
        c = lax.axis_index("c")
        s = lax.axis_index("s")
        pltpu.sync_copy(zeros_hbm.at[pl.ds(s * ROWS_A, ROWS_A)],
                        acc_spm.at[pl.ds(s * ROWS_A, ROWS_A)])
        pltpu.sync_copy(srcs_hbm.at[c, s], src_idx)
        pltpu.sync_copy(dsts_hbm.at[c, s], dst_idx)
        plsc.subcore_barrier()

        def step(j, carry):
            pltpu.sync_copy(t_hbm.at[src_idx.at[j]], rows)
            pltpu.sync_copy(rows, acc_spm.at[dst_idx.at[j]], add=True)
            return carry

        lax.fori_loop(0, CPT, step, 0)
        plsc.subcore_barrier()
        pltpu.sync_copy(acc_spm.at[pl.ds(s * ROWS_A, ROWS_A)],
                        out_hbm.at[c, pl.ds(s * ROWS_A, ROWS_A)])

    return functools.partial(
        pl.kernel,
        out_type=pltpu.HBM((NC, N_PAD, d), jnp.float32),
        mesh=_MESH,
        compiler_params=pltpu.CompilerParams(use_tc_tiling_on_sc=False),
        scratch_types=[
            pltpu.VMEM((CPT, CHUNK), jnp.int32),
            pltpu.VMEM((CPT, CHUNK), jnp.int32),
            pltpu.VMEM((CHUNK, d), jnp.float32),
            pltpu.VMEM_SHARED((N_PAD, d), jnp.float32),
            pltpu.SemaphoreType.DMA,
        ],
    )(body)


_agg8 = _make_agg(8)


def _mm_body(x_ref, w_ref, degp_ref, t1a_ref, t1b_ref, dinv_ref):
    deg = degp_ref[0, 0, :] + degp_ref[0, 1, :] + 1.0
    dinv = lax.rsqrt(deg)
    h = jnp.dot(x_ref[...].astype(jnp.bfloat16),
                w_ref[...].astype(jnp.bfloat16),
                preferred_element_type=jnp.float32)
    t1 = h * dinv[:, None]
    t1a_ref[...] = t1[:, :8]
    t1b_ref[...] = t1[:, 8:]
    dinv_ref[...] = dinv[:, None]


def _mid_body(pa_ref, pb_ref, t1a_ref, t1b_ref, dinv_ref, b1_ref, w2_ref,
              t2_ref):
    sa = pa_ref[0] + pa_ref[1] + t1a_ref[...]
    sb = pb_ref[0] + pb_ref[1] + t1b_ref[...]
    s = jnp.concatenate([sa, sb], axis=1)
    out1 = s * dinv_ref[...] + b1_ref[...]
    h1 = jnp.where(out1 > 0, out1, jnp.exp(out1) - 1.0)
    h2 = jnp.dot(h1, w2_ref[...], preferred_element_type=jnp.float32)
    t2_ref[...] = h2 * dinv_ref[...]


def _fin_body(q_ref, t2_ref, dinv_ref, b2_ref, out_ref):
    z = (q_ref[0] + q_ref[1] + t2_ref[...]) * dinv_ref[...] + b2_ref[...]
    lane = lax.broadcasted_iota(jnp.int32, (ROW_BLK, D_OUT_PAD), 1)
    zm = jnp.where(lane < D_OUT, z, -3e38)
    m = jnp.max(zm, axis=1, keepdims=True)
    lse = m + jnp.log(jnp.sum(jnp.exp(zm - m), axis=1, keepdims=True))
    out_ref[...] = z - lse


def _matmul_prep(x, w1, degp):
    return pl.pallas_call(
        _mm_body,
        grid=(GRID,),
        in_specs=[
            pl.BlockSpec((ROW_BLK, D_IN), lambda i: (i, 0)),
            pl.BlockSpec((D_IN, D_HID), lambda i: (0, 0)),
            pl.BlockSpec((1, NC, ROW_BLK), lambda i: (i, 0, 0)),
        ],
        out_specs=[
            pl.BlockSpec((ROW_BLK, 8), lambda i: (i, 0)),
            pl.BlockSpec((ROW_BLK, 8), lambda i: (i, 0)),
            pl.BlockSpec((ROW_BLK, 1), lambda i: (i, 0)),
        ],
        out_shape=[
            jax.ShapeDtypeStruct((N, 8), jnp.float32),
            jax.ShapeDtypeStruct((N, 8), jnp.float32),
            jax.ShapeDtypeStruct((N, 1), jnp.float32),
        ],
    )(x, w1, degp)


def _mid(pa, pb, t1a, t1b, dinv, b1, w2p):
    return pl.pallas_call(
        _mid_body,
        grid=(GRID,),
        in_specs=[
            pl.BlockSpec((NC, ROW_BLK, 8), lambda i: (0, i, 0)),
            pl.BlockSpec((NC, ROW_BLK, 8), lambda i: (0, i, 0)),
            pl.BlockSpec((ROW_BLK, 8), lambda i: (i, 0)),
            pl.BlockSpec((ROW_BLK, 8), lambda i: (i, 0)),
            pl.BlockSpec((ROW_BLK, 1), lambda i: (i, 0)),
            pl.BlockSpec((1, D_HID), lambda i: (0, 0)),
            pl.BlockSpec((D_HID, D_OUT_PAD), lambda i: (0, 0)),
        ],
        out_specs=pl.BlockSpec((ROW_BLK, D_OUT_PAD), lambda i: (i, 0)),
        out_shape=jax.ShapeDtypeStruct((N, D_OUT_PAD), jnp.float32),
    )(pa, pb, t1a, t1b, dinv, b1, w2p)


def _final(q, t2, dinv, b2p):
    return pl.pallas_call(
        _fin_body,
        grid=(GRID,),
        in_specs=[
            pl.BlockSpec((NC, ROW_BLK, D_OUT_PAD), lambda i: (0, i, 0)),
            pl.BlockSpec((ROW_BLK, D_OUT_PAD), lambda i: (i, 0)),
            pl.BlockSpec((ROW_BLK, 1), lambda i: (i, 0)),
            pl.BlockSpec((1, D_OUT_PAD), lambda i: (0, 0)),
        ],
        out_specs=pl.BlockSpec((ROW_BLK, D_OUT_PAD), lambda i: (i, 0)),
        out_shape=jax.ShapeDtypeStruct((N, D_OUT_PAD), jnp.float32),
    )(q, t2, dinv, b2p)


def kernel(x, edge_index, W1, b1, W2, b2):
    src = edge_index[0]
    dst = edge_index[1]
    pad = E_PAD - E
    src_p = jnp.concatenate([src, jnp.zeros((pad,), jnp.int32)])
    dst_p = jnp.concatenate([dst, jnp.full((pad,), N, jnp.int32)])
    srcs = src_p.reshape(NC, NS, CPT, CHUNK)
    dsts = dst_p.reshape(NC, NS, CPT, CHUNK)

    zeros8 = jnp.zeros((N_PAD, D_OUT_PAD), jnp.float32)
    w2p = jnp.pad(W2, ((0, 0), (0, D_OUT_PAD - D_OUT)))
    b1r = b1.reshape(1, D_HID)
    b2p = jnp.pad(b2, (0, D_OUT_PAD - D_OUT)).reshape(1, D_OUT_PAD)

    degp = _hist(dsts).reshape(NC, N_PAD)
    degp_b = degp[:, :N].reshape(NC, GRID, ROW_BLK).transpose(1, 0, 2)
    t1a, t1b, dinv = _matmul_prep(x, W1, degp_b)
    pa = _agg8(t1a, srcs, dsts, zeros8)
    pb = _agg8(t1b, srcs, dsts, zeros8)
    t2 = _mid(pa, pb, t1a, t1b, dinv, b1r, w2p)
    q = _agg8(t2, srcs, dsts, zeros8)
    out = _final(q, t2, dinv, b2p)
    return out[:, :D_OUT]


# speedup vs baseline: 34.4468x; 1.3728x over previous
import functools

import jax
import jax.numpy as jnp
from jax import lax
from jax.experimental import pallas as pl
from jax.experimental.pallas import tpu as pltpu
from jax.experimental.pallas import tpu_sc as plsc

N = 50000
E = 1600000
D_IN = 1433
D_HID = 16
D_OUT = 7
D_OUT_PAD = 8

NC = 2
NS = 16
NT = NC * NS

CHUNK = 128
KDEPTH = 4
CPT = 392
E_PAD = NT * CPT * CHUNK
NJ = CPT // KDEPTH

N_PAD = 50048
ROWS_A = N_PAD // NS
ROWS_LAST = N - (NS - 1) * ROWS_A

ROW_BLK = 400
GRID = N // ROW_BLK

_MESH = plsc.VectorSubcoreMesh(
    core_axis_name="c", subcore_axis_name="s", num_cores=NC, num_subcores=NS
)


ZBUF = ROWS_A + 8


def _hist_body(dsts_hbm, out_hbm, dst_idx, ones, zbuf, deg_acc, sem):
    c = lax.axis_index("c")
    s = lax.axis_index("s")
    for i in range(CHUNK // 16):
        ones[pl.ds(i * 16, 16)] = jnp.ones((16,), jnp.float32)
    for i in range(ZBUF // 16):
        zbuf[pl.ds(i * 16, 16)] = jnp.zeros((16,), jnp.float32)
    pltpu.sync_copy(zbuf.at[pl.ds(0, ROWS_A)],
                    deg_acc.at[pl.ds(s * ROWS_A, ROWS_A)])
    pltpu.sync_copy(dsts_hbm.at[c, s], dst_idx)
    plsc.subcore_barrier()

    def body(jj, carry):
        descs = [
            pltpu.async_copy(ones, deg_acc.at[dst_idx.at[jj * KDEPTH + b]],
                             sem, add=True)
            for b in range(KDEPTH)
        ]
        for d_ in descs:
            d_.wait()
        return carry

    lax.fori_loop(0, NJ, body, 0)
    plsc.subcore_barrier()
    pltpu.sync_copy(deg_acc.at[pl.ds(s * ROWS_A, ROWS_A)],
                    zbuf.at[pl.ds(0, ROWS_A)])
    pltpu.sync_copy(zbuf.at[pl.ds(0, ROWS_A)],
                    out_hbm.at[pl.ds(c * N_PAD + s * ROWS_A, ROWS_A)])


_hist = functools.partial(
    pl.kernel,
    out_type=pltpu.HBM((NC * N_PAD,), jnp.float32),
    mesh=_MESH,
    compiler_params=pltpu.CompilerParams(use_tc_tiling_on_sc=False),
    scratch_types=[
        pltpu.VMEM((CPT, CHUNK), jnp.int32),
        pltpu.VMEM((CHUNK,), jnp.float32),
        pltpu.VMEM((ZBUF,), jnp.float32),
        pltpu.VMEM_SHARED((N_PAD,), jnp.float32),
        pltpu.SemaphoreType.DMA,
    ],
)(_hist_body)


def _make_agg(d):
    def body(t_hbm, srcs_hbm, dsts_hbm, zeros_hbm, out_hbm,
             src_idx, dst_idx, rows, acc_spm, gsem, ssem):
        c = lax.axis_index("c")
        s = lax.axis_index("s")
        pltpu.sync_copy(zeros_hbm.at[pl.ds(s * ROWS_A, ROWS_A)],
                        acc_spm.at[pl.ds(s * ROWS_A, ROWS_A)])
        pltpu.sync_copy(srcs_hbm.at[c, s], src_idx)
        pltpu.sync_copy(dsts_hbm.at[c, s], dst_idx)
        plsc.subcore_barrier()

        def step(jj, carry):
            gd = [
                pltpu.async_copy(t_hbm.at[src_idx.at[jj * KDEPTH + b]],
                                 rows.at[b], gsem)
                for b in range(KDEPTH)
            ]
            for d_ in gd:
                d_.wait()
            sd = [
                pltpu.async_copy(rows.at[b],
                                 acc_spm.at[dst_idx.at[jj * KDEPTH + b]],
                                 ssem, add=True)
                for b in range(KDEPTH)
            ]
            for d_ in sd:
                d_.wait()
            return carry

        lax.fori_loop(0, NJ, step, 0)
        plsc.subcore_barrier()
        pltpu.sync_copy(acc_spm.at[pl.ds(s * ROWS_A, ROWS_A)],
                        out_hbm.at[c, pl.ds(s * ROWS_A, ROWS_A)])

    return functools.partial(
        pl.kernel,
        out_type=pltpu.HBM((NC, N_PAD, d), jnp.float32),
        mesh=_MESH,
        compiler_params=pltpu.CompilerParams(use_tc_tiling_on_sc=False),
        scratch_types=[
            pltpu.VMEM((CPT, CHUNK), jnp.int32),
            pltpu.VMEM((CPT, CHUNK), jnp.int32),
            pltpu.VMEM((KDEPTH, CHUNK, d), jnp.float32),
            pltpu.VMEM_SHARED((N_PAD, d), jnp.float32),
            pltpu.SemaphoreType.DMA,
            pltpu.SemaphoreType.DMA,
        ],
    )(body)


_agg8 = _make_agg(8)


def _mm_body(x_ref, w_ref, degp_ref, t1a_ref, t1b_ref, dinv_ref):
    deg = degp_ref[0, 0, :] + degp_ref[0, 1, :] + 1.0
    dinv = lax.rsqrt(deg)
    h = jnp.dot(x_ref[...].astype(jnp.bfloat16),
                w_ref[...].astype(jnp.bfloat16),
                preferred_element_type=jnp.float32)
    t1 = h * dinv[:, None]
    t1a_ref[...] = t1[:, :8]
    t1b_ref[...] = t1[:, 8:]
    dinv_ref[...] = dinv[:, None]


def _mid_body(pa_ref, pb_ref, t1a_ref, t1b_ref, dinv_ref, b1_ref, w2_ref,
              t2_ref):
    sa = pa_ref[0] + pa_ref[1] + t1a_ref[...]
    sb = pb_ref[0] + pb_ref[1] + t1b_ref[...]
    s = jnp.concatenate([sa, sb], axis=1)
    out1 = s * dinv_ref[...] + b1_ref[...]
    h1 = jnp.where(out1 > 0, out1, jnp.exp(out1) - 1.0)
    h2 = jnp.dot(h1, w2_ref[...], preferred_element_type=jnp.float32)
    t2_ref[...] = h2 * dinv_ref[...]


def _fin_body(q_ref, t2_ref, dinv_ref, b2_ref, out_ref):
    z = (q_ref[0] + q_ref[1] + t2_ref[...]) * dinv_ref[...] + b2_ref[...]
    lane = lax.broadcasted_iota(jnp.int32, (ROW_BLK, D_OUT_PAD), 1)
    zm = jnp.where(lane < D_OUT, z, -3e38)
    m = jnp.max(zm, axis=1, keepdims=True)
    lse = m + jnp.log(jnp.sum(jnp.exp(zm - m), axis=1, keepdims=True))
    out_ref[...] = z - lse


def _matmul_prep(x, w1, degp):
    return pl.pallas_call(
        _mm_body,
        grid=(GRID,),
        in_specs=[
            pl.BlockSpec((ROW_BLK, D_IN), lambda i: (i, 0)),
            pl.BlockSpec((D_IN, D_HID), lambda i: (0, 0)),
            pl.BlockSpec((1, NC, ROW_BLK), lambda i: (i, 0, 0)),
        ],
        out_specs=[
            pl.BlockSpec((ROW_BLK, 8), lambda i: (i, 0)),
            pl.BlockSpec((ROW_BLK, 8), lambda i: (i, 0)),
            pl.BlockSpec((ROW_BLK, 1), lambda i: (i, 0)),
        ],
        out_shape=[
            jax.ShapeDtypeStruct((N, 8), jnp.float32),
            jax.ShapeDtypeStruct((N, 8), jnp.float32),
            jax.ShapeDtypeStruct((N, 1), jnp.float32),
        ],
    )(x, w1, degp)


def _mid(pa, pb, t1a, t1b, dinv, b1, w2p):
    return pl.pallas_call(
        _mid_body,
        grid=(GRID,),
        in_specs=[
            pl.BlockSpec((NC, ROW_BLK, 8), lambda i: (0, i, 0)),
            pl.BlockSpec((NC, ROW_BLK, 8), lambda i: (0, i, 0)),
            pl.BlockSpec((ROW_BLK, 8), lambda i: (i, 0)),
            pl.BlockSpec((ROW_BLK, 8), lambda i: (i, 0)),
            pl.BlockSpec((ROW_BLK, 1), lambda i: (i, 0)),
            pl.BlockSpec((1, D_HID), lambda i: (0, 0)),
            pl.BlockSpec((D_HID, D_OUT_PAD), lambda i: (0, 0)),
        ],
        out_specs=pl.BlockSpec((ROW_BLK, D_OUT_PAD), lambda i: (i, 0)),
        out_shape=jax.ShapeDtypeStruct((N, D_OUT_PAD), jnp.float32),
    )(pa, pb, t1a, t1b, dinv, b1, w2p)


def _final(q, t2, dinv, b2p):
    return pl.pallas_call(
        _fin_body,
        grid=(GRID,),
        in_specs=[
            pl.BlockSpec((NC, ROW_BLK, D_OUT_PAD), lambda i: (0, i, 0)),
            pl.BlockSpec((ROW_BLK, D_OUT_PAD), lambda i: (i, 0)),
            pl.BlockSpec((ROW_BLK, 1), lambda i: (i, 0)),
            pl.BlockSpec((1, D_OUT_PAD), lambda i: (0, 0)),
        ],
        out_specs=pl.BlockSpec((ROW_BLK, D_OUT_PAD), lambda i: (i, 0)),
        out_shape=jax.ShapeDtypeStruct((N, D_OUT_PAD), jnp.float32),
    )(q, t2, dinv, b2p)


def kernel(x, edge_index, W1, b1, W2, b2):
    src = edge_index[0]
    dst = edge_index[1]
    pad = E_PAD - E
    src_p = jnp.concatenate([src, jnp.zeros((pad,), jnp.int32)])
    dst_p = jnp.concatenate([dst, jnp.full((pad,), N, jnp.int32)])
    srcs = src_p.reshape(NC, NS, CPT, CHUNK)
    dsts = dst_p.reshape(NC, NS, CPT, CHUNK)

    zeros8 = jnp.zeros((N_PAD, D_OUT_PAD), jnp.float32)
    w2p = jnp.pad(W2, ((0, 0), (0, D_OUT_PAD - D_OUT)))
    b1r = b1.reshape(1, D_HID)
    b2p = jnp.pad(b2, (0, D_OUT_PAD - D_OUT)).reshape(1, D_OUT_PAD)

    degp = _hist(dsts).reshape(NC, N_PAD)
    degp_b = degp[:, :N].reshape(NC, GRID, ROW_BLK).transpose(1, 0, 2)
    t1a, t1b, dinv = _matmul_prep(x, W1, degp_b)
    pa = _agg8(t1a, srcs, dsts, zeros8)
    pb = _agg8(t1b, srcs, dsts, zeros8)
    t2 = _mid(pa, pb, t1a, t1b, dinv, b1r, w2p)
    q = _agg8(t2, srcs, dsts, zeros8)
    out = _final(q, t2, dinv, b2p)
    return out[:, :D_OUT]
